# baseline (speedup 1.0000x reference)
import jax
import jax.numpy as jnp
from jax import lax
from jax.experimental import pallas as pl

_TOP_K = 1000
_CONF = 0.01
_B = 4
_N0 = 4096
_K = 4096
_W = 64
_ROWS = 512
_NSTEP = _N0 // _ROWS
_PADK = 1024


def _wins(va, pa, vb, pb):
    return (va > vb) | ((va == vb) & (pa < pb))


def _bitonic_stage(v, p, idx, j, desc, n):
    if j >= 128:
        nch = n // j
        pv = jnp.concatenate([v[..., (c ^ 1) * j:((c ^ 1) + 1) * j] for c in range(nch)], axis=-1)
        pp = jnp.concatenate([p[..., (c ^ 1) * j:((c ^ 1) + 1) * j] for c in range(nch)], axis=-1)
    else:
        vf = jnp.concatenate([v[..., j:], v[..., :j]], axis=-1)
        vb = jnp.concatenate([v[..., n - j:], v[..., :n - j]], axis=-1)
        pf = jnp.concatenate([p[..., j:], p[..., :j]], axis=-1)
        pb = jnp.concatenate([p[..., n - j:], p[..., :n - j]], axis=-1)
        lo = (idx & j) == 0
        pv = jnp.where(lo, vf, vb)
        pp = jnp.where(lo, pf, pb)
    bit_lo = (idx & j) == 0
    self_wins = _wins(v, p, pv, pp)
    keep = (self_wins == bit_lo) == desc
    return jnp.where(keep, v, pv), jnp.where(keep, p, pp)


def _reduce_body(x_ref, mv_ref, mp_ref):
    x = x_ref[...]
    ntile = _K // 128
    vm = x[:, :, 0:128]
    it = jnp.zeros((1, _ROWS, 128), jnp.int32)
    for t in range(1, ntile):
        xt = x[:, :, t * 128:(t + 1) * 128]
        gt = xt > vm
        it = jnp.where(gt, t, it)
        vm = jnp.where(gt, xt, vm)
    m = jnp.max(vm, axis=-1)
    lane = lax.broadcasted_iota(jnp.int32, (1, _ROWS, 128), 2)
    gidx = (it << 7) | lane
    hit = jnp.where(vm == m[..., None], gidx, _K)
    mi = jnp.min(hit, axis=-1)

    r = pl.program_id(0) % _NSTEP
    lidx = lax.broadcasted_iota(jnp.int32, (1, _ROWS), 1)
    v = jnp.where(m > _CONF, m, -jnp.inf)
    p = ((lidx + r * _ROWS) << 12) | mi
    k = 2
    while k <= _ROWS:
        if k < _ROWS:
            desc = (lidx & k) == 0
        else:
            desc = (r & 1) == 0
        j = k // 2
        while j >= 1:
            v, p = _bitonic_stage(v, p, lidx, j, desc, _ROWS)
            j //= 2
        k *= 2
    mv_ref[...] = v[:, None, :]
    mp_ref[...] = p[:, None, :]


def _topk_body(mv_ref, mp_ref, conf_ref, x0_ref, y0_ref, x1_ref, y1_ref):
    v = mv_ref[...]
    p = mp_ref[...]
    idx = lax.broadcasted_iota(jnp.int32, (_B, _N0), 1)

    for k in (1024, 2048):
        desc = (idx & k) == 0
        j = k // 2
        while j >= 1:
            v, p = _bitonic_stage(v, p, idx, j, desc, _N0)
            j //= 2

    for half in (2048, 1024):
        lw = _wins(v[:, :half], p[:, :half], v[:, half:], p[:, half:])
        v = jnp.where(lw, v[:, :half], v[:, half:])
        p = jnp.where(lw, p[:, :half], p[:, half:])
    idx2 = lax.broadcasted_iota(jnp.int32, (_B, _PADK), 1)
    j = 512
    while j >= 1:
        v, p = _bitonic_stage(v, p, idx2, j, True, _PADK)
        j //= 2

    sidx = p >> 12
    sanch = p & (_N0 - 1)
    valid = v > _CONF
    inv = jnp.float32(1.0 / (_W - 1))
    fz = jnp.float32(0.0)
    conf_ref[...] = jnp.where(valid, v, fz)
    x0_ref[...] = jnp.where(valid, (sidx & (_W - 1)).astype(jnp.float32) * inv, fz)
    y0_ref[...] = jnp.where(valid, (sidx >> 6).astype(jnp.float32) * inv, fz)
    x1_ref[...] = jnp.where(valid, (sanch & (_W - 1)).astype(jnp.float32) * inv, fz)
    y1_ref[...] = jnp.where(valid, (sanch >> 6).astype(jnp.float32) * inv, fz)


def kernel(anchor_probs):
    B, N0, K = anchor_probs.shape
    mv3, mp3 = pl.pallas_call(
        _reduce_body,
        grid=(B * _NSTEP,),
        in_specs=[pl.BlockSpec((1, _ROWS, K), lambda g: (g // _NSTEP, g % _NSTEP, 0))],
        out_specs=[
            pl.BlockSpec((1, 1, _ROWS), lambda g: (g, 0, 0)),
            pl.BlockSpec((1, 1, _ROWS), lambda g: (g, 0, 0)),
        ],
        out_shape=[
            jax.ShapeDtypeStruct((B * _NSTEP, 1, _ROWS), jnp.float32),
            jax.ShapeDtypeStruct((B * _NSTEP, 1, _ROWS), jnp.int32),
        ],
    )(anchor_probs)
    maxv = mv3.reshape(B, N0)
    maxp = mp3.reshape(B, N0)

    conf, x0, y0, x1, y1 = pl.pallas_call(
        _topk_body,
        out_shape=[jax.ShapeDtypeStruct((_B, _PADK), jnp.float32)] * 5,
    )(maxv, maxp)

    conf = conf[:, :_TOP_K]
    mkpts0 = jnp.stack([x0[:, :_TOP_K], y0[:, :_TOP_K]], axis=-1).reshape(-1, 2)
    mkpts1 = jnp.stack([x1[:, :_TOP_K], y1[:, :_TOP_K]], axis=-1).reshape(-1, 2)
    mconf = conf.reshape(-1)
    b_ids = jnp.broadcast_to(jnp.arange(B)[:, None], (B, _TOP_K)).reshape(-1)
    return (mkpts0, mkpts1, mconf, b_ids)


# speedup vs baseline: 6.6994x; 6.6994x over previous
import jax
import jax.numpy as jnp
from jax import lax
from jax.experimental import pallas as pl

_TOP_K = 1000
_CONF = 0.01
_B = 4
_N0 = 4096
_K = 4096
_W = 64
_ROWS = 512
_NSTEP = _N0 // _ROWS
_LANES = 512


def _reduce_body(x_ref, mv_ref, mi_ref):
    x = x_ref[...]
    ntile = _K // 128
    vm = x[:, :, 0:128]
    it = jnp.zeros((1, _ROWS, 128), jnp.int32)
    for t in range(1, ntile):
        xt = x[:, :, t * 128:(t + 1) * 128]
        gt = xt > vm
        it = jnp.where(gt, t, it)
        vm = jnp.where(gt, xt, vm)
    m = jnp.max(vm, axis=-1)
    lane = lax.broadcasted_iota(jnp.int32, (1, _ROWS, 128), 2)
    g = (it << 7) | lane
    hit = jnp.where(vm == m[..., None], g, _K)
    mi = jnp.min(hit, axis=-1)
    mv_ref[...] = m[:, None, :]
    mi_ref[...] = mi[:, None, :]


def _wins(va, pa, vb, pb):
    return (va > vb) | ((va == vb) & (pa < pb))


def _stage(v, p, j, desc, l_idx, s_idx):
    nsub = v.shape[1]
    if j >= _LANES:
        js = j // _LANES
        order = [c ^ 1 for c in range(nsub // js)]
        pv = jnp.concatenate([v[:, c * js:(c + 1) * js] for c in order], axis=1)
        pp = jnp.concatenate([p[:, c * js:(c + 1) * js] for c in order], axis=1)
        bit_lo = (s_idx & js) == 0
    elif j >= 128:
        order = [c ^ 1 for c in range(_LANES // j)]
        pv = jnp.concatenate([v[..., c * j:(c + 1) * j] for c in order], axis=-1)
        pp = jnp.concatenate([p[..., c * j:(c + 1) * j] for c in order], axis=-1)
        bit_lo = (l_idx & j) == 0
    else:
        bit_lo = (l_idx & j) == 0
        vf = jnp.concatenate([v[..., j:], v[..., :j]], axis=-1)
        vb = jnp.concatenate([v[..., _LANES - j:], v[..., :_LANES - j]], axis=-1)
        pf = jnp.concatenate([p[..., j:], p[..., :j]], axis=-1)
        pb = jnp.concatenate([p[..., _LANES - j:], p[..., :_LANES - j]], axis=-1)
        pv = jnp.where(bit_lo, vf, vb)
        pp = jnp.where(bit_lo, pf, pb)
    self_wins = _wins(v, p, pv, pp)
    keep = (self_wins == bit_lo) == desc
    return jnp.where(keep, v, pv), jnp.where(keep, p, pp)


def _topk_body(mv_ref, mi_ref, conf_ref, x0_ref, y0_ref, x1_ref, y1_ref):
    m = mv_ref[...]
    anch = mi_ref[...]
    shp = m.shape
    l_idx = lax.broadcasted_iota(jnp.int32, shp, 2)
    s_idx = lax.broadcasted_iota(jnp.int32, shp, 1)
    v = jnp.where(m > _CONF, m, -jnp.inf)
    p = (((s_idx << 9) | l_idx) << 12) | anch

    def desc_mask(k):
        if k < _LANES:
            return (l_idx[:, :v.shape[1]] & k) == 0
        return (s_idx[:, :v.shape[1]] & (k // _LANES)) == 0

    k = 2
    while k <= 2048:
        desc = desc_mask(k)
        j = k // 2
        while j >= 1:
            v, p = _stage(v, p, j, desc, l_idx[:, :v.shape[1]], s_idx[:, :v.shape[1]])
            j //= 2
        k *= 2

    lw = _wins(v[:, :4], p[:, :4], v[:, 4:], p[:, 4:])
    v = jnp.where(lw, v[:, :4], v[:, 4:])
    p = jnp.where(lw, p[:, :4], p[:, 4:])
    j = 1024
    while j >= 1:
        v, p = _stage(v, p, j, True, l_idx[:, :4], s_idx[:, :4])
        j //= 2

    v = v[:, :2]
    p = p[:, :2]
    sidx = p >> 12
    sanch = p & (_N0 - 1)
    valid = v > _CONF
    inv = jnp.float32(1.0 / (_W - 1))
    fz = jnp.float32(0.0)
    conf_ref[...] = jnp.where(valid, v, fz)
    x0_ref[...] = jnp.where(valid, (sidx & (_W - 1)).astype(jnp.float32) * inv, fz)
    y0_ref[...] = jnp.where(valid, ((sidx >> 6) & (_W - 1)).astype(jnp.float32) * inv, fz)
    x1_ref[...] = jnp.where(valid, (sanch & (_W - 1)).astype(jnp.float32) * inv, fz)
    y1_ref[...] = jnp.where(valid, (sanch >> 6).astype(jnp.float32) * inv, fz)


def kernel(anchor_probs):
    B, N0, K = anchor_probs.shape
    mv3, mi3 = pl.pallas_call(
        _reduce_body,
        grid=(B * _NSTEP,),
        in_specs=[pl.BlockSpec((1, _ROWS, K), lambda g: (g // _NSTEP, g % _NSTEP, 0))],
        out_specs=[
            pl.BlockSpec((1, 1, _ROWS), lambda g: (g, 0, 0)),
            pl.BlockSpec((1, 1, _ROWS), lambda g: (g, 0, 0)),
        ],
        out_shape=[
            jax.ShapeDtypeStruct((B * _NSTEP, 1, _ROWS), jnp.float32),
            jax.ShapeDtypeStruct((B * _NSTEP, 1, _ROWS), jnp.int32),
        ],
    )(anchor_probs)
    maxv = mv3.reshape(B, _NSTEP, _ROWS)
    maxi = mi3.reshape(B, _NSTEP, _ROWS)

    conf, x0, y0, x1, y1 = pl.pallas_call(
        _topk_body,
        out_shape=[jax.ShapeDtypeStruct((_B, 2, _LANES), jnp.float32)] * 5,
    )(maxv, maxi)

    conf = conf.reshape(_B, 2 * _LANES)[:, :_TOP_K]
    x0 = x0.reshape(_B, 2 * _LANES)[:, :_TOP_K]
    y0 = y0.reshape(_B, 2 * _LANES)[:, :_TOP_K]
    x1 = x1.reshape(_B, 2 * _LANES)[:, :_TOP_K]
    y1 = y1.reshape(_B, 2 * _LANES)[:, :_TOP_K]
    mkpts0 = jnp.stack([x0, y0], axis=-1).reshape(-1, 2)
    mkpts1 = jnp.stack([x1, y1], axis=-1).reshape(-1, 2)
    mconf = conf.reshape(-1)
    b_ids = jnp.broadcast_to(jnp.arange(B)[:, None], (B, _TOP_K)).reshape(-1)
    return (mkpts0, mkpts1, mconf, b_ids)


# speedup vs baseline: 6.7360x; 1.0055x over previous
import jax
import jax.numpy as jnp
from jax import lax
from jax.experimental import pallas as pl

_TOP_K = 1000
_CONF = 0.01
_B = 4
_N0 = 4096
_K = 4096
_W = 64
_ROWS = 512
_NSTEP = _N0 // _ROWS
_LANES = 512


def _reduce_body(x_ref, mv_ref, mi_ref):
    x = x_ref[...]
    ntile = _K // 128
    vm = x[:, :, 0:128]
    it = jnp.zeros((1, _ROWS, 128), jnp.int32)
    for t in range(1, ntile):
        xt = x[:, :, t * 128:(t + 1) * 128]
        gt = xt > vm
        it = jnp.where(gt, t, it)
        vm = jnp.where(gt, xt, vm)
    m = jnp.max(vm, axis=-1)
    lane = lax.broadcasted_iota(jnp.int32, (1, _ROWS, 128), 2)
    g = (it << 7) | lane
    hit = jnp.where(vm == m[..., None], g, _K)
    mi = jnp.min(hit, axis=-1)
    mv_ref[...] = m[:, None, :]
    mi_ref[...] = mi[:, None, :]


def _wins(va, pa, vb, pb):
    return (va > vb) | ((va == vb) & (pa < pb))


def _cx(vA, pA, vB, pB):
    aw = _wins(vA, pA, vB, pB)
    w = (jnp.where(aw, vA, vB), jnp.where(aw, pA, pB))
    l = (jnp.where(aw, vB, vA), jnp.where(aw, pB, pA))
    return w, l


def _sub_stage(v, p, j, k, truncate=False):
    js = j // _LANES
    nch = v.shape[1] // js
    vout = [None] * nch
    pout = [None] * nch
    for c in range(nch // 2):
        a, b = 2 * c, 2 * c + 1
        vA, pA = v[:, a * js:(a + 1) * js], p[:, a * js:(a + 1) * js]
        vB, pB = v[:, b * js:(b + 1) * js], p[:, b * js:(b + 1) * js]
        (wv, wp), (lv, lp) = _cx(vA, pA, vB, pB)
        desc = True if k >= 4096 else ((a * js) & (k // _LANES)) == 0
        if desc:
            vout[a], pout[a], vout[b], pout[b] = wv, wp, lv, lp
        else:
            vout[a], pout[a], vout[b], pout[b] = lv, lp, wv, wp
    if truncate:
        vout, pout = vout[:nch // 2], pout[:nch // 2]
    if len(vout) == 1:
        return vout[0], pout[0]
    return jnp.concatenate(vout, axis=1), jnp.concatenate(pout, axis=1)


def _lane_chunk_stage(v, p, j, k, s_idx):
    nch = _LANES // j
    vout = [None] * nch
    pout = [None] * nch
    for c in range(nch // 2):
        a, b = 2 * c, 2 * c + 1
        vA, pA = v[..., a * j:(a + 1) * j], p[..., a * j:(a + 1) * j]
        vB, pB = v[..., b * j:(b + 1) * j], p[..., b * j:(b + 1) * j]
        (wv, wp), (lv, lp) = _cx(vA, pA, vB, pB)
        if k >= 4096:
            desc = True
        elif k >= _LANES:
            desc = (s_idx[..., :j] & (k // _LANES)) == 0
        else:
            desc = ((a * j) & k) == 0
        if desc is True:
            vout[a], pout[a], vout[b], pout[b] = wv, wp, lv, lp
        elif desc is False:
            vout[a], pout[a], vout[b], pout[b] = lv, lp, wv, wp
        else:
            vout[a] = jnp.where(desc, wv, lv)
            pout[a] = jnp.where(desc, wp, lp)
            vout[b] = jnp.where(desc, lv, wv)
            pout[b] = jnp.where(desc, lp, wp)
    return jnp.concatenate(vout, axis=-1), jnp.concatenate(pout, axis=-1)


def _roll_stage(v, p, j, k, l_idx, s_idx):
    bit_lo = (l_idx & j) == 0
    if k >= 4096:
        desc = True
    elif k >= _LANES:
        desc = (s_idx & (k // _LANES)) == 0
    else:
        desc = (l_idx & k) == 0
    vf = jnp.concatenate([v[..., j:], v[..., :j]], axis=-1)
    vb = jnp.concatenate([v[..., _LANES - j:], v[..., :_LANES - j]], axis=-1)
    pf = jnp.concatenate([p[..., j:], p[..., :j]], axis=-1)
    pb = jnp.concatenate([p[..., _LANES - j:], p[..., :_LANES - j]], axis=-1)
    pv = jnp.where(bit_lo, vf, vb)
    pp = jnp.where(bit_lo, pf, pb)
    self_wins = _wins(v, p, pv, pp)
    if desc is True:
        keep = self_wins == bit_lo
    else:
        keep = (self_wins == bit_lo) == desc
    return jnp.where(keep, v, pv), jnp.where(keep, p, pp)


def _topk_body(mv_ref, mi_ref, conf_ref, x0_ref, y0_ref, x1_ref, y1_ref):
    m = mv_ref[...]
    anch = mi_ref[...]
    shp = m.shape
    l_idx = lax.broadcasted_iota(jnp.int32, shp, 2)
    s_idx = lax.broadcasted_iota(jnp.int32, shp, 1)
    v = jnp.where(m > _CONF, m, -jnp.inf)
    p = (((s_idx << 9) | l_idx) << 12) | anch

    def stage(v, p, j, k, truncate=False):
        if j >= _LANES:
            return _sub_stage(v, p, j, k, truncate)
        sub = v.shape[1]
        if j >= 128:
            return _lane_chunk_stage(v, p, j, k, s_idx[:, :sub])
        return _roll_stage(v, p, j, k, l_idx[:, :sub], s_idx[:, :sub])

    k = 2
    while k <= 2048:
        j = k // 2
        while j >= 1:
            v, p = stage(v, p, j, k)
            j //= 2
        k *= 2

    v, p = stage(v, p, 2048, 4096, truncate=True)
    v, p = stage(v, p, 1024, 4096, truncate=True)
    j = 512
    while j >= 1:
        v, p = stage(v, p, j, 4096)
        j //= 2

    v = jnp.concatenate([v[:, 0], v[:, 1]], axis=-1)
    p = jnp.concatenate([p[:, 0], p[:, 1]], axis=-1)
    sidx = p >> 12
    sanch = p & (_N0 - 1)
    valid = v > _CONF
    inv = jnp.float32(1.0 / (_W - 1))
    fz = jnp.float32(0.0)
    conf_ref[...] = jnp.where(valid, v, fz)[:, :_TOP_K]
    x0_ref[...] = jnp.where(valid, (sidx & (_W - 1)).astype(jnp.float32) * inv, fz)[:, :_TOP_K]
    y0_ref[...] = jnp.where(valid, ((sidx >> 6) & (_W - 1)).astype(jnp.float32) * inv, fz)[:, :_TOP_K]
    x1_ref[...] = jnp.where(valid, (sanch & (_W - 1)).astype(jnp.float32) * inv, fz)[:, :_TOP_K]
    y1_ref[...] = jnp.where(valid, (sanch >> 6).astype(jnp.float32) * inv, fz)[:, :_TOP_K]


def kernel(anchor_probs):
    B, N0, K = anchor_probs.shape
    mv3, mi3 = pl.pallas_call(
        _reduce_body,
        grid=(B * _NSTEP,),
        in_specs=[pl.BlockSpec((1, _ROWS, K), lambda g: (g // _NSTEP, g % _NSTEP, 0))],
        out_specs=[
            pl.BlockSpec((1, 1, _ROWS), lambda g: (g, 0, 0)),
            pl.BlockSpec((1, 1, _ROWS), lambda g: (g, 0, 0)),
        ],
        out_shape=[
            jax.ShapeDtypeStruct((B * _NSTEP, 1, _ROWS), jnp.float32),
            jax.ShapeDtypeStruct((B * _NSTEP, 1, _ROWS), jnp.int32),
        ],
    )(anchor_probs)
    maxv = mv3.reshape(B, _NSTEP, _ROWS)
    maxi = mi3.reshape(B, _NSTEP, _ROWS)

    conf, x0, y0, x1, y1 = pl.pallas_call(
        _topk_body,
        out_shape=[jax.ShapeDtypeStruct((_B, _TOP_K), jnp.float32)] * 5,
    )(maxv, maxi)

    mkpts0 = jnp.stack([x0, y0], axis=-1).reshape(-1, 2)
    mkpts1 = jnp.stack([x1, y1], axis=-1).reshape(-1, 2)
    mconf = conf.reshape(-1)
    b_ids = jnp.broadcast_to(jnp.arange(B)[:, None], (B, _TOP_K)).reshape(-1)
    return (mkpts0, mkpts1, mconf, b_ids)


# speedup vs baseline: 7.0315x; 1.0439x over previous
import jax
import jax.numpy as jnp
from jax import lax
from jax.experimental import pallas as pl
from jax.experimental.pallas import tpu as pltpu

_TOP_K = 1000
_CONF = 0.01
_B = 4
_N0 = 4096
_K = 4096
_W = 64
_ROWS = 512
_NSTEP = _N0 // _ROWS
_LANES = 512


def _reduce_rows(x):
    ntile = _K // 128
    vm = x[:, :, 0:128]
    it = jnp.zeros((1, _ROWS, 128), jnp.int32)
    for t in range(1, ntile):
        xt = x[:, :, t * 128:(t + 1) * 128]
        gt = xt > vm
        it = jnp.where(gt, t, it)
        vm = jnp.where(gt, xt, vm)
    m = jnp.max(vm, axis=-1)
    lane = lax.broadcasted_iota(jnp.int32, (1, _ROWS, 128), 2)
    g = (it << 7) | lane
    hit = jnp.where(vm == m[..., None], g, _K)
    mi = jnp.min(hit, axis=-1)
    return m, mi


def _wins(va, pa, vb, pb):
    return (va > vb) | ((va == vb) & (pa < pb))


def _cx(vA, pA, vB, pB):
    aw = _wins(vA, pA, vB, pB)
    w = (jnp.where(aw, vA, vB), jnp.where(aw, pA, pB))
    l = (jnp.where(aw, vB, vA), jnp.where(aw, pB, pA))
    return w, l


def _sub_stage(v, p, j, k, truncate=False):
    js = j // _LANES
    nch = v.shape[1] // js
    vout = [None] * nch
    pout = [None] * nch
    for c in range(nch // 2):
        a, b = 2 * c, 2 * c + 1
        vA, pA = v[:, a * js:(a + 1) * js], p[:, a * js:(a + 1) * js]
        vB, pB = v[:, b * js:(b + 1) * js], p[:, b * js:(b + 1) * js]
        (wv, wp), (lv, lp) = _cx(vA, pA, vB, pB)
        desc = True if k >= 4096 else ((a * js) & (k // _LANES)) == 0
        if desc:
            vout[a], pout[a], vout[b], pout[b] = wv, wp, lv, lp
        else:
            vout[a], pout[a], vout[b], pout[b] = lv, lp, wv, wp
    if truncate:
        vout, pout = vout[:nch // 2], pout[:nch // 2]
    if len(vout) == 1:
        return vout[0], pout[0]
    return jnp.concatenate(vout, axis=1), jnp.concatenate(pout, axis=1)


def _lane_chunk_stage(v, p, j, k, s_idx):
    nch = _LANES // j
    vout = [None] * nch
    pout = [None] * nch
    for c in range(nch // 2):
        a, b = 2 * c, 2 * c + 1
        vA, pA = v[..., a * j:(a + 1) * j], p[..., a * j:(a + 1) * j]
        vB, pB = v[..., b * j:(b + 1) * j], p[..., b * j:(b + 1) * j]
        (wv, wp), (lv, lp) = _cx(vA, pA, vB, pB)
        if k >= 4096:
            desc = True
        elif k >= _LANES:
            desc = (s_idx[..., :j] & (k // _LANES)) == 0
        else:
            desc = ((a * j) & k) == 0
        if desc is True:
            vout[a], pout[a], vout[b], pout[b] = wv, wp, lv, lp
        elif desc is False:
            vout[a], pout[a], vout[b], pout[b] = lv, lp, wv, wp
        else:
            vout[a] = jnp.where(desc, wv, lv)
            pout[a] = jnp.where(desc, wp, lp)
            vout[b] = jnp.where(desc, lv, wv)
            pout[b] = jnp.where(desc, lp, wp)
    return jnp.concatenate(vout, axis=-1), jnp.concatenate(pout, axis=-1)


def _roll_stage(v, p, j, k, l_idx, s_idx):
    bit_lo = (l_idx & j) == 0
    if k >= 4096:
        desc = True
    elif k >= _LANES:
        desc = (s_idx & (k // _LANES)) == 0
    else:
        desc = (l_idx & k) == 0
    vf = jnp.concatenate([v[..., j:], v[..., :j]], axis=-1)
    vb = jnp.concatenate([v[..., _LANES - j:], v[..., :_LANES - j]], axis=-1)
    pf = jnp.concatenate([p[..., j:], p[..., :j]], axis=-1)
    pb = jnp.concatenate([p[..., _LANES - j:], p[..., :_LANES - j]], axis=-1)
    pv = jnp.where(bit_lo, vf, vb)
    pp = jnp.where(bit_lo, pf, pb)
    self_wins = _wins(v, p, pv, pp)
    if desc is True:
        keep = self_wins == bit_lo
    else:
        keep = (self_wins == bit_lo) == desc
    return jnp.where(keep, v, pv), jnp.where(keep, p, pp)


def _topk_compute(m, anch, conf_ref, x0_ref, y0_ref, x1_ref, y1_ref):
    shp = m.shape
    l_idx = lax.broadcasted_iota(jnp.int32, shp, 2)
    s_idx = lax.broadcasted_iota(jnp.int32, shp, 1)
    v = jnp.where(m > _CONF, m, -jnp.inf)
    p = (((s_idx << 9) | l_idx) << 12) | anch

    def stage(v, p, j, k, truncate=False):
        if j >= _LANES:
            return _sub_stage(v, p, j, k, truncate)
        sub = v.shape[1]
        if j >= 128:
            return _lane_chunk_stage(v, p, j, k, s_idx[:, :sub])
        return _roll_stage(v, p, j, k, l_idx[:, :sub], s_idx[:, :sub])

    k = 2
    while k <= 2048:
        j = k // 2
        while j >= 1:
            v, p = stage(v, p, j, k)
            j //= 2
        k *= 2

    v, p = stage(v, p, 2048, 4096, truncate=True)
    v, p = stage(v, p, 1024, 4096, truncate=True)
    j = 512
    while j >= 1:
        v, p = stage(v, p, j, 4096)
        j //= 2

    v = jnp.concatenate([v[:, 0], v[:, 1]], axis=-1)
    p = jnp.concatenate([p[:, 0], p[:, 1]], axis=-1)
    sidx = p >> 12
    sanch = p & (_N0 - 1)
    valid = v > _CONF
    inv = jnp.float32(1.0 / (_W - 1))
    fz = jnp.float32(0.0)
    conf_ref[...] = jnp.where(valid, v, fz)[:, :_TOP_K]
    x0_ref[...] = jnp.where(valid, (sidx & (_W - 1)).astype(jnp.float32) * inv, fz)[:, :_TOP_K]
    y0_ref[...] = jnp.where(valid, ((sidx >> 6) & (_W - 1)).astype(jnp.float32) * inv, fz)[:, :_TOP_K]
    x1_ref[...] = jnp.where(valid, (sanch & (_W - 1)).astype(jnp.float32) * inv, fz)[:, :_TOP_K]
    y1_ref[...] = jnp.where(valid, (sanch >> 6).astype(jnp.float32) * inv, fz)[:, :_TOP_K]


def _fused_body(x_ref, conf_ref, x0_ref, y0_ref, x1_ref, y1_ref, mv_acc, mi_acc):
    g = pl.program_id(0)
    b = g // _NSTEP
    r = g % _NSTEP
    m, mi = _reduce_rows(x_ref[...])
    mv_acc[b, r] = m[0]
    mi_acc[b, r] = mi[0]

    @pl.when(g == _B * _NSTEP - 1)
    def _():
        _topk_compute(mv_acc[...], mi_acc[...],
                      conf_ref, x0_ref, y0_ref, x1_ref, y1_ref)


def kernel(anchor_probs):
    B, N0, K = anchor_probs.shape
    out_spec = pl.BlockSpec((_B, _TOP_K), lambda g: (0, 0))
    conf, x0, y0, x1, y1 = pl.pallas_call(
        _fused_body,
        grid=(B * _NSTEP,),
        in_specs=[pl.BlockSpec((1, _ROWS, K), lambda g: (g // _NSTEP, g % _NSTEP, 0))],
        out_specs=[out_spec] * 5,
        out_shape=[jax.ShapeDtypeStruct((_B, _TOP_K), jnp.float32)] * 5,
        scratch_shapes=[
            pltpu.VMEM((_B, _NSTEP, _ROWS), jnp.float32),
            pltpu.VMEM((_B, _NSTEP, _ROWS), jnp.int32),
        ],
    )(anchor_probs)

    mkpts0 = jnp.stack([x0, y0], axis=-1).reshape(-1, 2)
    mkpts1 = jnp.stack([x1, y1], axis=-1).reshape(-1, 2)
    mconf = conf.reshape(-1)
    b_ids = jnp.broadcast_to(jnp.arange(B)[:, None], (B, _TOP_K)).reshape(-1)
    return (mkpts0, mkpts1, mconf, b_ids)
